# Initial kernel scaffold; baseline (speedup 1.0000x reference)
import jax
import jax.numpy as jnp
from jax.experimental import pallas as pl


def kernel(x, edge_index, edge_attr, We0, be0, W10, b10, W20, b20, We1, be1, W11, b11, W21, b21, We2, be2, W12, b12, W22, b22, Wout, bout):
    raise NotImplementedError("write your pallas kernel here")


# speedup vs baseline: 2.9028x; 2.9028x over previous
import functools

import jax
import jax.numpy as jnp
from jax import lax
from jax.experimental import pallas as pl
from jax.experimental.pallas import tpu as pltpu
from jax.experimental.pallas import tpu_sc as plsc

N = 10000
E = 320000
D = 128
ED = 16
H = 128

NC = 2
NS = 16
NW = NC * NS
C = 128
NCHUNK = E // C
KMAX = -(-NCHUNK // NW)
RPT = 640
RPT_LAST = N - 15 * RPT


def _sc_message_pass(h, e, src2, dst2):
    mesh = plsc.VectorSubcoreMesh(core_axis_name="c", subcore_axis_name="s")

    @functools.partial(
        pl.kernel,
        mesh=mesh,
        out_type=jax.ShapeDtypeStruct((NC, N, D), jnp.float32),
        scratch_types=[
            pltpu.VMEM((C,), jnp.int32),
            pltpu.VMEM((C,), jnp.int32),
            pltpu.VMEM((C, D), jnp.float32),
            pltpu.VMEM((C, D), jnp.float32),
            pltpu.VMEM_SHARED((N, D), jnp.float32),
            pltpu.SemaphoreType.DMA,
        ],
    )
    def body(h_hbm, e_hbm, src_hbm, dst_hbm, out_hbm,
             src_v, dst_v, rows_v, e_v, acc, sem):
        cid = lax.axis_index("c")
        sid = lax.axis_index("s")
        wid = cid * NS + sid

        r0 = sid * RPT

        @pl.when(sid < NS - 1)
        def _():
            pltpu.sync_copy(h_hbm.at[pl.ds(r0, RPT)], acc.at[pl.ds(r0, RPT)])

        @pl.when(sid == NS - 1)
        def _():
            pltpu.sync_copy(h_hbm.at[pl.ds(r0, RPT_LAST)],
                            acc.at[pl.ds(r0, RPT_LAST)])

        plsc.subcore_barrier()

        def step(k, carry):
            chunk = wid + NW * k

            @pl.when(chunk < NCHUNK)
            def _():
                base = chunk * C
                pltpu.sync_copy(src_hbm.at[pl.ds(base, C)], src_v)
                pltpu.sync_copy(dst_hbm.at[pl.ds(base, C)], dst_v)
                pltpu.async_copy(h_hbm.at[src_v], rows_v, sem).wait()
                pltpu.sync_copy(e_hbm.at[pl.ds(base, C)], e_v)

                def crow(r, c2):
                    for j in range(D // 16):
                        s = pl.ds(j * 16, 16)
                        rows_v[r, s] = jnp.maximum(rows_v[r, s] + e_v[r, s],
                                                   0.0)
                    return c2

                lax.fori_loop(0, C, crow, 0)
                pltpu.sync_copy(rows_v, acc.at[dst_v], add=True)

            return carry

        lax.fori_loop(0, KMAX, step, 0)
        plsc.subcore_barrier()

        @pl.when(sid < NS - 1)
        def _():
            pltpu.sync_copy(acc.at[pl.ds(r0, RPT)],
                            out_hbm.at[cid, pl.ds(r0, RPT)])

        @pl.when(sid == NS - 1)
        def _():
            pltpu.sync_copy(acc.at[pl.ds(r0, RPT_LAST)],
                            out_hbm.at[cid, pl.ds(r0, RPT_LAST)])

    return body(h, e, src2, dst2)


BE = 4000


def _edge_linear(edge_attr, W_stack, b_stack):
    def body(ea_ref, w_ref, b_ref, o0, o1, o2):
        ea = ea_ref[...]
        outs = (o0, o1, o2)
        for l in range(3):
            outs[l][...] = jnp.dot(ea, w_ref[l],
                                   preferred_element_type=jnp.float32) + b_ref[l]

    return pl.pallas_call(
        body,
        grid=(E // BE,),
        in_specs=[
            pl.BlockSpec((BE, ED), lambda i: (i, 0)),
            pl.BlockSpec((3, ED, D), lambda i: (0, 0, 0)),
            pl.BlockSpec((3, D), lambda i: (0, 0)),
        ],
        out_specs=[pl.BlockSpec((BE, D), lambda i: (i, 0))] * 3,
        out_shape=[jax.ShapeDtypeStruct((E, D), jnp.float32)] * 3,
    )(edge_attr, W_stack, b_stack)


BN = 2000


def _mlp(h, agg, W1, b1, W2, b2):
    def body(h_ref, a_ref, w1, b1r, w2, b2r, out_ref):
        z = a_ref[0] + a_ref[1] - h_ref[...]
        z1 = jnp.maximum(jnp.dot(z, w1[...],
                                 preferred_element_type=jnp.float32) + b1r[...], 0.0)
        z2 = jnp.dot(z1, w2[...], preferred_element_type=jnp.float32) + b2r[...]
        out_ref[...] = jnp.maximum(z2, 0.0)

    return pl.pallas_call(
        body,
        grid=(N // BN,),
        in_specs=[
            pl.BlockSpec((BN, D), lambda i: (i, 0)),
            pl.BlockSpec((NC, BN, D), lambda i: (0, i, 0)),
            pl.BlockSpec((D, H), lambda i: (0, 0)),
            pl.BlockSpec((H,), lambda i: (0,)),
            pl.BlockSpec((H, H), lambda i: (0, 0)),
            pl.BlockSpec((H,), lambda i: (0,)),
        ],
        out_specs=pl.BlockSpec((BN, H), lambda i: (i, 0)),
        out_shape=jax.ShapeDtypeStruct((N, H), jnp.float32),
    )(h, agg, W1, b1, W2, b2)


def _mlp_head(h, agg, W1, b1, W2, b2, Wout, bout):
    def body(h_ref, a_ref, w1, b1r, w2, b2r, wo, bo, out_ref):
        z = a_ref[0] + a_ref[1] - h_ref[...]
        z1 = jnp.maximum(jnp.dot(z, w1[...],
                                 preferred_element_type=jnp.float32) + b1r[...], 0.0)
        z2 = jnp.dot(z1, w2[...], preferred_element_type=jnp.float32) + b2r[...]
        hf = jnp.maximum(z2, 0.0)
        logit = jnp.dot(hf, wo[...], preferred_element_type=jnp.float32) + bo[...]
        out_ref[...] = jax.nn.sigmoid(logit)

    return pl.pallas_call(
        body,
        grid=(N // BN,),
        in_specs=[
            pl.BlockSpec((BN, D), lambda i: (i, 0)),
            pl.BlockSpec((NC, BN, D), lambda i: (0, i, 0)),
            pl.BlockSpec((D, H), lambda i: (0, 0)),
            pl.BlockSpec((H,), lambda i: (0,)),
            pl.BlockSpec((H, H), lambda i: (0, 0)),
            pl.BlockSpec((H,), lambda i: (0,)),
            pl.BlockSpec((H, 1), lambda i: (0, 0)),
            pl.BlockSpec((1,), lambda i: (0,)),
        ],
        out_specs=pl.BlockSpec((BN, 1), lambda i: (i, 0)),
        out_shape=jax.ShapeDtypeStruct((N, 1), jnp.float32),
    )(h, agg, W1, b1, W2, b2, Wout, bout)


def kernel(x, edge_index, edge_attr,
           We0, be0, W10, b10, W20, b20,
           We1, be1, W11, b11, W21, b21,
           We2, be2, W12, b12, W22, b22,
           Wout, bout):
    W_stack = jnp.stack([We0, We1, We2])
    b_stack = jnp.stack([be0, be1, be2])
    e0, e1, e2 = _edge_linear(edge_attr, W_stack, b_stack)

    src2 = edge_index[0]
    dst2 = edge_index[1]

    h = x
    agg = _sc_message_pass(h, e0, src2, dst2)
    h = _mlp(h, agg, W10, b10, W20, b20)
    agg = _sc_message_pass(h, e1, src2, dst2)
    h = _mlp(h, agg, W11, b11, W21, b21)
    agg = _sc_message_pass(h, e2, src2, dst2)
    out = _mlp_head(h, agg, W12, b12, W22, b22, Wout, bout)
    return out.reshape(N)


# speedup vs baseline: 3.8376x; 1.3220x over previous
import functools

import jax
import jax.numpy as jnp
from jax import lax
from jax.experimental import pallas as pl
from jax.experimental.pallas import tpu as pltpu
from jax.experimental.pallas import tpu_sc as plsc

N = 10000
E = 320000
D = 128
ED = 16
H = 128

NC = 2
NS = 16
NW = NC * NS
C = 64
NCHUNK = E // C
KMAX = -(-NCHUNK // NW)
RPT = 640
RPT_LAST = N - 15 * RPT


def _sc_message_pass(h, e, src2, dst2):
    mesh = plsc.VectorSubcoreMesh(core_axis_name="c", subcore_axis_name="s")

    @functools.partial(
        pl.kernel,
        mesh=mesh,
        out_type=jax.ShapeDtypeStruct((NC, N, D), jnp.float32),
        scratch_types=[
            pltpu.VMEM((C,), jnp.int32),
            pltpu.VMEM((C,), jnp.int32),
            pltpu.VMEM((C, D), jnp.float32),
            pltpu.VMEM((C, D), jnp.float32),
            pltpu.VMEM((C,), jnp.int32),
            pltpu.VMEM((C,), jnp.int32),
            pltpu.VMEM((C, D), jnp.float32),
            pltpu.VMEM((C, D), jnp.float32),
            pltpu.VMEM_SHARED((N, D), jnp.float32),
            pltpu.SemaphoreType.DMA,
            pltpu.SemaphoreType.DMA,
        ],
    )
    def body(h_hbm, e_hbm, src_hbm, dst_hbm, out_hbm,
             src0, dst0, rows0, ev0, src1, dst1, rows1, ev1,
             acc, sem0, sem1):
        bufs = ((src0, dst0, rows0, ev0, sem0),
                (src1, dst1, rows1, ev1, sem1))
        cid = lax.axis_index("c")
        sid = lax.axis_index("s")
        wid = cid * NS + sid

        r0 = sid * RPT

        @pl.when(sid < NS - 1)
        def _():
            pltpu.sync_copy(h_hbm.at[pl.ds(r0, RPT)], acc.at[pl.ds(r0, RPT)])

        @pl.when(sid == NS - 1)
        def _():
            pltpu.sync_copy(h_hbm.at[pl.ds(r0, RPT_LAST)],
                            acc.at[pl.ds(r0, RPT_LAST)])

        plsc.subcore_barrier()

        def start(k, b):
            sv, dv, rv, ev, sem = bufs[b]
            chunk = wid + NW * k

            @pl.when(chunk < NCHUNK)
            def _():
                base = chunk * C
                pltpu.sync_copy(src_hbm.at[pl.ds(base, C)], sv)
                pltpu.sync_copy(dst_hbm.at[pl.ds(base, C)], dv)
                pltpu.async_copy(h_hbm.at[sv], rv, sem)
                pltpu.async_copy(e_hbm.at[pl.ds(base, C)], ev, sem)

        def finish(k, b):
            sv, dv, rv, ev, sem = bufs[b]
            chunk = wid + NW * k

            @pl.when(chunk < NCHUNK)
            def _():
                base = chunk * C
                pltpu.make_async_copy(h_hbm.at[sv], rv, sem).wait()
                pltpu.make_async_copy(e_hbm.at[pl.ds(base, C)], ev,
                                      sem).wait()

                def crow(r, c2):
                    for t in range(2):
                        for j in range(D // 16):
                            s = pl.ds(j * 16, 16)
                            rv[2 * r + t, s] = jnp.maximum(
                                rv[2 * r + t, s] + ev[2 * r + t, s], 0.0)
                    return c2

                lax.fori_loop(0, C // 2, crow, 0)
                pltpu.sync_copy(rv, acc.at[dv], add=True)

        start(0, 0)

        def step(k2, carry):
            k = 2 * k2
            start(k + 1, 1)
            finish(k, 0)
            start(k + 2, 0)
            finish(k + 1, 1)
            return carry

        lax.fori_loop(0, (KMAX + 1) // 2, step, 0)
        plsc.subcore_barrier()

        @pl.when(sid < NS - 1)
        def _():
            pltpu.sync_copy(acc.at[pl.ds(r0, RPT)],
                            out_hbm.at[cid, pl.ds(r0, RPT)])

        @pl.when(sid == NS - 1)
        def _():
            pltpu.sync_copy(acc.at[pl.ds(r0, RPT_LAST)],
                            out_hbm.at[cid, pl.ds(r0, RPT_LAST)])

    return body(h, e, src2, dst2)


BE = 4000


def _edge_linear(edge_attr, W_stack, b_stack):
    def body(ea_ref, w_ref, b_ref, o0, o1, o2):
        ea = ea_ref[...]
        outs = (o0, o1, o2)
        for l in range(3):
            outs[l][...] = jnp.dot(ea, w_ref[l],
                                   preferred_element_type=jnp.float32) + b_ref[l]

    return pl.pallas_call(
        body,
        grid=(E // BE,),
        in_specs=[
            pl.BlockSpec((BE, ED), lambda i: (i, 0)),
            pl.BlockSpec((3, ED, D), lambda i: (0, 0, 0)),
            pl.BlockSpec((3, D), lambda i: (0, 0)),
        ],
        out_specs=[pl.BlockSpec((BE, D), lambda i: (i, 0))] * 3,
        out_shape=[jax.ShapeDtypeStruct((E, D), jnp.float32)] * 3,
    )(edge_attr, W_stack, b_stack)


BN = 2000


def _mlp(h, agg, W1, b1, W2, b2):
    def body(h_ref, a_ref, w1, b1r, w2, b2r, out_ref):
        z = a_ref[0] + a_ref[1] - h_ref[...]
        z1 = jnp.maximum(jnp.dot(z, w1[...],
                                 preferred_element_type=jnp.float32) + b1r[...], 0.0)
        z2 = jnp.dot(z1, w2[...], preferred_element_type=jnp.float32) + b2r[...]
        out_ref[...] = jnp.maximum(z2, 0.0)

    return pl.pallas_call(
        body,
        grid=(N // BN,),
        in_specs=[
            pl.BlockSpec((BN, D), lambda i: (i, 0)),
            pl.BlockSpec((NC, BN, D), lambda i: (0, i, 0)),
            pl.BlockSpec((D, H), lambda i: (0, 0)),
            pl.BlockSpec((H,), lambda i: (0,)),
            pl.BlockSpec((H, H), lambda i: (0, 0)),
            pl.BlockSpec((H,), lambda i: (0,)),
        ],
        out_specs=pl.BlockSpec((BN, H), lambda i: (i, 0)),
        out_shape=jax.ShapeDtypeStruct((N, H), jnp.float32),
    )(h, agg, W1, b1, W2, b2)


def _mlp_head(h, agg, W1, b1, W2, b2, Wout, bout):
    def body(h_ref, a_ref, w1, b1r, w2, b2r, wo, bo, out_ref):
        z = a_ref[0] + a_ref[1] - h_ref[...]
        z1 = jnp.maximum(jnp.dot(z, w1[...],
                                 preferred_element_type=jnp.float32) + b1r[...], 0.0)
        z2 = jnp.dot(z1, w2[...], preferred_element_type=jnp.float32) + b2r[...]
        hf = jnp.maximum(z2, 0.0)
        logit = jnp.dot(hf, wo[...], preferred_element_type=jnp.float32) + bo[...]
        out_ref[...] = jax.nn.sigmoid(logit)

    return pl.pallas_call(
        body,
        grid=(N // BN,),
        in_specs=[
            pl.BlockSpec((BN, D), lambda i: (i, 0)),
            pl.BlockSpec((NC, BN, D), lambda i: (0, i, 0)),
            pl.BlockSpec((D, H), lambda i: (0, 0)),
            pl.BlockSpec((H,), lambda i: (0,)),
            pl.BlockSpec((H, H), lambda i: (0, 0)),
            pl.BlockSpec((H,), lambda i: (0,)),
            pl.BlockSpec((H, 1), lambda i: (0, 0)),
            pl.BlockSpec((1,), lambda i: (0,)),
        ],
        out_specs=pl.BlockSpec((BN, 1), lambda i: (i, 0)),
        out_shape=jax.ShapeDtypeStruct((N, 1), jnp.float32),
    )(h, agg, W1, b1, W2, b2, Wout, bout)


def kernel(x, edge_index, edge_attr,
           We0, be0, W10, b10, W20, b20,
           We1, be1, W11, b11, W21, b21,
           We2, be2, W12, b12, W22, b22,
           Wout, bout):
    W_stack = jnp.stack([We0, We1, We2])
    b_stack = jnp.stack([be0, be1, be2])
    e0, e1, e2 = _edge_linear(edge_attr, W_stack, b_stack)

    src2 = edge_index[0]
    dst2 = edge_index[1]

    h = x
    agg = _sc_message_pass(h, e0, src2, dst2)
    h = _mlp(h, agg, W10, b10, W20, b20)
    agg = _sc_message_pass(h, e1, src2, dst2)
    h = _mlp(h, agg, W11, b11, W21, b21)
    agg = _sc_message_pass(h, e2, src2, dst2)
    out = _mlp_head(h, agg, W12, b12, W22, b22, Wout, bout)
    return out.reshape(N)
